# baseline (speedup 1.0000x reference)
import jax
import jax.numpy as jnp
from jax import lax
from jax.experimental import pallas as pl
from jax.experimental.pallas import tpu as pltpu
from jax.experimental.pallas import tpu_sc as plsc

EMBED_DIM = 64
WINDOW = 128
SUPER = 4
DEPTH = 1
NBUF = 2
N_WORKERS = 32


def kernel(token_ids, embedding):
    batch, hist = token_ids.shape
    n_idx = batch * hist
    n_super = n_idx // (SUPER * WINDOW)
    supers_per_worker = n_super // N_WORKERS
    n_rounds = supers_per_worker // NBUF

    idx = token_ids.reshape(n_super, SUPER * WINDOW).astype(jnp.int32)

    mesh = plsc.VectorSubcoreMesh(core_axis_name="core", subcore_axis_name="subcore")

    @pl.kernel(
        out_type=jax.ShapeDtypeStruct((n_super, SUPER * WINDOW, EMBED_DIM), embedding.dtype),
        mesh=mesh,
        scratch_types=[
            pltpu.VMEM((supers_per_worker, SUPER * WINDOW), jnp.int32),
            pltpu.VMEM((NBUF, SUPER * WINDOW, EMBED_DIM), jnp.float32),
            pltpu.SemaphoreType.DMA((NBUF,)),
            pltpu.SemaphoreType.DMA((NBUF,)),
        ],
        compiler_params=pltpu.CompilerParams(use_tc_tiling_on_sc=False),
    )
    def gather_kernel(table_hbm, idx_hbm, out_hbm, idx_v, rows_v, gsem, wsem):
        wid = lax.axis_index("subcore") * 2 + lax.axis_index("core")
        super0 = wid * supers_per_worker

        def start_gather(c, s):
            pltpu.async_copy(table_hbm.at[idx_v.at[c]], rows_v.at[s], gsem.at[s])

        def wait_gather(s):
            pltpu.make_async_copy(table_hbm.at[idx_v.at[0]], rows_v.at[s], gsem.at[s]).wait()

        def start_write(c, s):
            pltpu.async_copy(rows_v.at[s], out_hbm.at[super0 + c], wsem.at[s])

        def wait_write(s):
            pltpu.make_async_copy(out_hbm.at[super0], rows_v.at[s], wsem.at[s]).wait()

        pltpu.sync_copy(idx_hbm.at[pl.ds(super0, supers_per_worker)], idx_v)

        for s in range(DEPTH):
            start_gather(s, s)

        for j in range(NBUF):
            wait_gather(j)
            start_write(j, j)
            s_n = (j + DEPTH) % NBUF
            if j >= NBUF - DEPTH:
                wait_write(s_n)
            start_gather(j + DEPTH, s_n)

        def round_body(r, _):
            c0 = r * NBUF
            for j in range(NBUF):
                wait_gather(j)
                start_write(c0 + j, j)
                s_n = (j + DEPTH) % NBUF
                wait_write(s_n)
                start_gather(c0 + j + DEPTH, s_n)
            return _

        lax.fori_loop(1, n_rounds - 1, round_body, 0)

        c0 = (n_rounds - 1) * NBUF
        for j in range(NBUF):
            wait_gather(j)
            start_write(c0 + j, j)
            if j < NBUF - DEPTH:
                s_n = (j + DEPTH) % NBUF
                wait_write(s_n)
                start_gather(c0 + j + DEPTH, s_n)

        for s in range(NBUF):
            wait_write(s)

    out = gather_kernel(embedding, idx)
    return out.reshape(batch, hist, EMBED_DIM)
